# Initial kernel scaffold; baseline (speedup 1.0000x reference)
import jax
import jax.numpy as jnp
from jax.experimental import pallas as pl


def kernel(x0, x1, x2, perm):
    raise NotImplementedError("write your pallas kernel here")


# speedup vs baseline: 2.8438x; 2.8438x over previous
import functools

import jax
import jax.numpy as jnp
from jax import lax
from jax.experimental import pallas as pl
from jax.experimental.pallas import tpu as pltpu
from jax.experimental.pallas import tpu_sc as plsc


def _scramble_rows(x0f, idx2, n_rows, d, n_workers, rows_per_worker, chunk):
    n_chunks = rows_per_worker // chunk
    mesh = plsc.VectorSubcoreMesh(core_axis_name="c", subcore_axis_name="s")

    @functools.partial(
        pl.kernel,
        mesh=mesh,
        out_type=jax.ShapeDtypeStruct((n_rows, d), jnp.float32),
        scratch_types=[
            pltpu.VMEM((rows_per_worker,), jnp.int32),
            pltpu.VMEM((2, chunk, d), jnp.float32),
            pltpu.SemaphoreType.DMA,
            pltpu.SemaphoreType.DMA,
        ],
    )
    def body(x0_hbm, idx_hbm, out_hbm, idx_v, bufs, g0, g1):
        wid = lax.axis_index("s") * 2 + lax.axis_index("c")
        base = wid * rows_per_worker
        pltpu.sync_copy(idx_hbm.at[wid], idx_v)
        gsems = (g0, g1)

        def fire(ch, slot):
            pltpu.async_copy(
                x0_hbm.at[idx_v.at[pl.ds(ch * chunk, chunk)]],
                bufs.at[slot],
                gsems[slot],
            )

        def wait(slot):
            pltpu.make_async_copy(
                x0_hbm.at[idx_v.at[pl.ds(0, chunk)]],
                bufs.at[slot],
                gsems[slot],
            ).wait()

        fire(0, 0)
        fire(1, 1)

        def step(i, _):
            for b in range(2):
                ch = 2 * i + b
                wait(b)
                pltpu.sync_copy(
                    bufs.at[b], out_hbm.at[pl.ds(base + ch * chunk, chunk)]
                )

                @pl.when(ch + 2 < n_chunks)
                def _():
                    fire(ch + 2, b)

            return 0

        lax.fori_loop(0, n_chunks // 2, step, 0)

    return body(x0f, idx2)


def kernel(x0, x1, x2, perm):
    b, s, d = x0.shape
    n = b * s
    n_workers = 32
    rows_per_worker = n // n_workers
    chunk = 16

    x0f = x0.reshape(n, d)
    idx = (
        jnp.arange(b, dtype=jnp.int32)[:, None] * s + perm[None, :].astype(jnp.int32)
    ).reshape(n_workers, rows_per_worker)

    outf = _scramble_rows(x0f, idx, n, d, n_workers, rows_per_worker, chunk)
    return outf.reshape(b, s, d), x1, x2
